# baseline (speedup 1.0000x reference)
import jax
import jax.numpy as jnp
from jax import lax
from jax.experimental import pallas as pl
from jax.experimental.pallas import tpu as pltpu
from jax.experimental.pallas import tpu_sc as plsc

_NC, _NS, _L = 2, 16, 16
_NW = _NC * _NS
_B = 16384
_BPW = _B // _NW
_D = 32
_G = 128 // _D
_C = 128
_NCHUNK = _BPW // _C
_GLOBAL_MEAN = 3.5


def _body(in_hbm, ut_hbm, it_hbm, ub_hbm, ib_hbm, out_hbm,
          uidx, iidx, ugrp, igrp, urows, irows, ubias, ibias, acc, sem):
    wid = lax.axis_index("s") * _NC + lax.axis_index("c")
    base = wid * _BPW

    pltpu.sync_copy(in_hbm.at[0].at[pl.ds(base, _BPW)], uidx)
    pltpu.sync_copy(in_hbm.at[1].at[pl.ds(base, _BPW)], iidx)

    @pl.loop(0, _BPW // _L)
    def _g(b):
        sl = pl.ds(b * _L, _L)
        ugrp[sl] = jnp.right_shift(uidx[sl], 2)
        igrp[sl] = jnp.right_shift(iidx[sl], 2)

    for q in range(_NCHUNK):
        sl = pl.ds(q * _C, _C)
        copies = [
            pltpu.async_copy(ut_hbm.at[ugrp.at[sl]], urows, sem),
            pltpu.async_copy(it_hbm.at[igrp.at[sl]], irows, sem),
            pltpu.async_copy(ub_hbm.at[0].at[uidx.at[sl]], ubias, sem),
            pltpu.async_copy(ib_hbm.at[0].at[iidx.at[sl]], ibias, sem),
        ]
        for c in copies:
            c.wait()

        @pl.loop(0, _C // _L)
        def _blk(b):
            csl = pl.ds(b * _L, _L)
            gsl = pl.ds(q * _C + b * _L, _L)
            rows = lax.iota(jnp.int32, _L) + b * _L
            uoff = jnp.left_shift(jnp.bitwise_and(uidx[gsl], _G - 1), 5)
            ioff = jnp.left_shift(jnp.bitwise_and(iidx[gsl], _G - 1), 5)
            accv = ubias[csl] + ibias[csl] + _GLOBAL_MEAN
            for d in range(_D):
                u = plsc.load_gather(urows, [rows, uoff + d])
                v = plsc.load_gather(irows, [rows, ioff + d])
                accv = accv + u * v
            acc[gsl] = accv

    pltpu.sync_copy(acc, out_hbm.at[pl.ds(base, _BPW)])


def kernel(inputs, user_table, item_table, user_bias_table, item_bias_table):
    inputs_t = inputs.T.astype(jnp.int32)
    ut_g = jnp.concatenate([user_table[m::4] for m in range(4)], axis=1)
    it_g = jnp.concatenate([item_table[m::4] for m in range(4)], axis=1)
    mesh = plsc.VectorSubcoreMesh(core_axis_name="c", subcore_axis_name="s")
    run = pl.kernel(
        _body,
        out_type=jax.ShapeDtypeStruct((_B,), jnp.float32),
        mesh=mesh,
        scratch_types=[
            pltpu.VMEM((_BPW,), jnp.int32),
            pltpu.VMEM((_BPW,), jnp.int32),
            pltpu.VMEM((_BPW,), jnp.int32),
            pltpu.VMEM((_BPW,), jnp.int32),
            pltpu.VMEM((_C, 128), jnp.float32),
            pltpu.VMEM((_C, 128), jnp.float32),
            pltpu.VMEM((_C,), jnp.float32),
            pltpu.VMEM((_C,), jnp.float32),
            pltpu.VMEM((_BPW,), jnp.float32),
            pltpu.SemaphoreType.DMA,
        ],
        compiler_params=pltpu.CompilerParams(needs_layout_passes=False),
    )
    out = run(inputs_t, ut_g, it_g, user_bias_table.T, item_bias_table.T)
    return out.reshape(_B, 1)


# speedup vs baseline: 10.7809x; 10.7809x over previous
import jax
import jax.numpy as jnp
from jax import lax
from jax.experimental import pallas as pl
from jax.experimental.pallas import tpu as pltpu
from jax.experimental.pallas import tpu_sc as plsc

_NC, _NS, _L = 2, 16, 16
_NW = _NC * _NS
_B = 16384
_BPW = _B // _NW
_D = 32
_G = 128 // _D
_C = 128
_NCHUNK = _BPW // _C
_GLOBAL_MEAN = 3.5


def _body(in_hbm, ut_hbm, it_hbm, ub_hbm, ib_hbm, out_hbm,
          uidx, iidx, ugrp, igrp, urows, irows, ubias, ibias, acc, sem):
    wid = lax.axis_index("s") * _NC + lax.axis_index("c")
    base = wid * _BPW

    pltpu.sync_copy(in_hbm.at[0].at[pl.ds(base, _BPW)], uidx)
    pltpu.sync_copy(in_hbm.at[1].at[pl.ds(base, _BPW)], iidx)

    @pl.loop(0, _BPW // _L)
    def _g(b):
        sl = pl.ds(b * _L, _L)
        ugrp[sl] = jnp.right_shift(uidx[sl], 2)
        igrp[sl] = jnp.right_shift(iidx[sl], 2)

    for q in range(_NCHUNK):
        sl = pl.ds(q * _C, _C)
        copies = [
            pltpu.async_copy(ut_hbm.at[ugrp.at[sl]], urows, sem),
            pltpu.async_copy(it_hbm.at[igrp.at[sl]], irows, sem),
            pltpu.async_copy(ub_hbm.at[0].at[uidx.at[sl]], ubias, sem),
            pltpu.async_copy(ib_hbm.at[0].at[iidx.at[sl]], ibias, sem),
        ]
        for c in copies:
            c.wait()

        @pl.loop(0, _C // _L)
        def _blk(b):
            csl = pl.ds(b * _L, _L)
            gsl = pl.ds(q * _C + b * _L, _L)
            rows = lax.iota(jnp.int32, _L) + b * _L
            uoff = jnp.left_shift(jnp.bitwise_and(uidx[gsl], _G - 1), 5)
            ioff = jnp.left_shift(jnp.bitwise_and(iidx[gsl], _G - 1), 5)
            accv = ubias[csl] + ibias[csl] + _GLOBAL_MEAN
            for d in range(_D):
                u = plsc.load_gather(urows, [rows, uoff + d])
                v = plsc.load_gather(irows, [rows, ioff + d])
                accv = accv + u * v
            acc[gsl] = accv

    pltpu.sync_copy(acc, out_hbm.at[pl.ds(base, _BPW)])


def kernel(inputs, user_table, item_table, user_bias_table, item_bias_table):
    inputs_t = inputs.T.astype(jnp.int32)
    def _group_rows(t):
        tt = jnp.swapaxes(t, 0, 1)
        return tt.reshape(_D, 250000, 4).transpose(1, 2, 0).reshape(250000, 128)

    ut_g = _group_rows(user_table)
    it_g = _group_rows(item_table)
    mesh = plsc.VectorSubcoreMesh(core_axis_name="c", subcore_axis_name="s")
    run = pl.kernel(
        _body,
        out_type=jax.ShapeDtypeStruct((_B,), jnp.float32),
        mesh=mesh,
        scratch_types=[
            pltpu.VMEM((_BPW,), jnp.int32),
            pltpu.VMEM((_BPW,), jnp.int32),
            pltpu.VMEM((_BPW,), jnp.int32),
            pltpu.VMEM((_BPW,), jnp.int32),
            pltpu.VMEM((_C, 128), jnp.float32),
            pltpu.VMEM((_C, 128), jnp.float32),
            pltpu.VMEM((_C,), jnp.float32),
            pltpu.VMEM((_C,), jnp.float32),
            pltpu.VMEM((_BPW,), jnp.float32),
            pltpu.SemaphoreType.DMA,
        ],
        compiler_params=pltpu.CompilerParams(needs_layout_passes=False),
    )
    out = run(inputs_t, ut_g, it_g, user_bias_table.T, item_bias_table.T)
    return out.reshape(_B, 1)
